# baseline (speedup 1.0000x reference)
import jax
import jax.numpy as jnp
from jax.experimental import pallas as pl
from jax.experimental.pallas import tpu as pltpu

_RB = 512
_N = 4
_S = 2048


def _mask_kernel(*refs):
    xs = refs[:_N]
    o_ref = refs[_N]
    i = pl.program_id(1)
    for k in range(_N):
        w = (k + 1) * _RB

        @pl.when(i == k)
        def _stripe(k=k, x_ref=xs[k], w=w):
            rows = k * _RB + jax.lax.broadcasted_iota(jnp.int32, (1, _RB, w), 1)
            cols = jax.lax.broadcasted_iota(jnp.int32, (1, _RB, w), 2)
            o_ref[:, :, :w] = jnp.where(cols > rows, jnp.float32(1.0), x_ref[...])
            if w < _S:
                o_ref[:, :, w:] = jnp.ones((1, _RB, _S - w), o_ref.dtype)


def kernel(x):
    B, S, _ = x.shape
    grid = (B, _N)

    def _in_map(k):
        def _map(b, i, k=k):
            b_eff = jnp.minimum(b + (i > k).astype(b.dtype), B - 1)
            return (b_eff, k, 0)

        return _map

    in_specs = [
        pl.BlockSpec((1, _RB, (k + 1) * _RB), _in_map(k)) for k in range(_N)
    ]
    return pl.pallas_call(
        _mask_kernel,
        grid=grid,
        in_specs=in_specs,
        out_specs=pl.BlockSpec((1, _RB, S), lambda b, i: (b, i, 0)),
        out_shape=jax.ShapeDtypeStruct(x.shape, x.dtype),
        compiler_params=pltpu.CompilerParams(
            dimension_semantics=("parallel", "arbitrary"),
        ),
    )(*([x] * _N))
